# baseline (speedup 1.0000x reference)
import functools
import math

import jax
import jax.numpy as jnp
from jax import lax
from jax.experimental import pallas as pl
from jax.experimental.pallas import tpu as pltpu
from jax.experimental.pallas import tpu_sc as plsc

N_ROWS = 2_000_000
OUT_COLS = 16
CHUNK = 2000
GROUPS = CHUNK // 16
NCHUNKS = N_ROWS // CHUNK
NUM_CORES = 2
NUM_SUBCORES = 16
NW = NUM_CORES * NUM_SUBCORES
SLOTS = (NCHUNKS + NW - 1) // NW
TAB_ROWS = 13


@functools.partial(
    pl.kernel,
    mesh=plsc.VectorSubcoreMesh(core_axis_name="c", subcore_axis_name="s"),
    out_type=jax.ShapeDtypeStruct((N_ROWS, OUT_COLS), jnp.float32),
    compiler_params=pltpu.CompilerParams(
        needs_layout_passes=False, use_tc_tiling_on_sc=False),
    scratch_types=[
        pltpu.VMEM((TAB_ROWS * 128,), jnp.float32),
        pltpu.VMEM((CHUNK, 4), jnp.int32),
        pltpu.VMEM((CHUNK, 4), jnp.int32),
        pltpu.VMEM((CHUNK, OUT_COLS), jnp.float32),
        pltpu.VMEM((CHUNK, OUT_COLS), jnp.float32),
        pltpu.SemaphoreType.DMA,
        pltpu.SemaphoreType.DMA,
        pltpu.SemaphoreType.DMA,
        pltpu.SemaphoreType.DMA,
    ],
)
def _sc_embed(note_hbm, tab_hbm, out_hbm, tab_v, in_a, in_b, out_a, out_b,
              in_sem_a, in_sem_b, out_sem_a, out_sem_b):
    wid = lax.axis_index("s") * NUM_CORES + lax.axis_index("c")
    pltpu.sync_copy(tab_hbm, tab_v)

    iota = lax.iota(jnp.int32, 16)
    ins = (in_a, in_b)
    outs = (out_a, out_b)
    in_sems = (in_sem_a, in_sem_b)
    out_sems = (out_sem_a, out_sem_b)
    csts = [jnp.full((16,), c, jnp.int32) for c in range(OUT_COLS)]

    def cid(i):
        return wid + i * NW

    def start_in(i):
        b = i % 2
        pltpu.async_copy(
            note_hbm.at[pl.ds(cid(i) * CHUNK, CHUNK)], ins[b], in_sems[b])

    def wait_in(i):
        b = i % 2
        pltpu.make_async_copy(
            note_hbm.at[pl.ds(0, CHUNK)], ins[b], in_sems[b]).wait()

    def start_out(i):
        b = i % 2
        pltpu.async_copy(
            outs[b], out_hbm.at[pl.ds(cid(i) * CHUNK, CHUNK)], out_sems[b])

    def wait_out(i):
        b = i % 2
        pltpu.make_async_copy(
            outs[b], out_hbm.at[pl.ds(0, CHUNK)], out_sems[b]).wait()

    def compute(i):
        inv = ins[i % 2]
        outv = outs[i % 2]

        def group(j, carry):
            rvec = j * 16 + iota
            pos = plsc.load_gather(inv, [rvec, csts[0]])
            pit = plsc.load_gather(inv, [rvec, csts[1]])
            lng = plsc.load_gather(inv, [rvec, csts[2]])
            vel = plsc.load_gather(inv, [rvec, csts[3]])
            cols = [plsc.load_gather(tab_v, [pit + (c * 128)])
                    for c in range(8)]
            cols.append(plsc.load_gather(tab_v, [vel + (10 * 128)]))
            cols.append(plsc.load_gather(tab_v, [lng + (11 * 128)]))
            cols.append(plsc.load_gather(tab_v, [pos + (8 * 128)]))
            cols.append(plsc.load_gather(tab_v, [pos + (9 * 128)]))
            pf = plsc.load_gather(tab_v, [pos + (12 * 128)])
            cols.extend([pf, pf, pf, pf])
            for c in range(OUT_COLS):
                plsc.store_scatter(outv, [rvec, csts[c]], cols[c])
            return carry

        lax.fori_loop(0, GROUPS, group, 0)

    start_in(0)
    for i in range(SLOTS):
        def body(i=i):
            if i + 1 < SLOTS:
                if NCHUNKS - (i + 1) * NW >= NW:
                    start_in(i + 1)
                else:
                    pl.when(cid(i + 1) < NCHUNKS)(lambda: start_in(i + 1))
            wait_in(i)
            if i >= 2:
                wait_out(i - 2)
            compute(i)
            start_out(i)

        if NCHUNKS - i * NW >= NW:
            body()
        else:
            pl.when(cid(i) < NCHUNKS)(body)

    wait_out(SLOTS - 2)
    wait_out(SLOTS - 1)


def kernel(note_rep, length, resolution, time_sig_num, time_sig_den,
           note_table, octave_table):
    g = jnp.arange(128, dtype=jnp.int32)
    gf = g.astype(jnp.float32)
    nt = note_table[g % 12]
    oc = octave_table[jnp.clip(g // 12 - 1, 0, octave_table.shape[0] - 1)]
    two_pi = 2.0 * math.pi
    w_beat = two_pi / resolution
    w_measure = two_pi / (resolution * (time_sig_num / (time_sig_den / 4)))
    w_melody = two_pi / (resolution * length)
    l_sin = jnp.sin(jnp.sin(jnp.sin(gf * w_beat) * w_measure) * w_melody)
    l_cos = jnp.cos(jnp.cos(jnp.cos(gf * w_beat) * w_measure) * w_melody)
    l_vel = (g / 127).astype(jnp.float32)
    l_len = (g / length).astype(jnp.float32)
    tab = jnp.concatenate(
        [nt.T, oc.T, l_sin[None], l_cos[None], l_vel[None], l_len[None],
         gf[None]], axis=0)
    return _sc_embed(note_rep, tab.reshape(-1))


# speedup vs baseline: 1.1213x; 1.1213x over previous
import functools
import math

import jax
import jax.numpy as jnp
from jax import lax
from jax.experimental import pallas as pl
from jax.experimental.pallas import tpu as pltpu
from jax.experimental.pallas import tpu_sc as plsc

N_ROWS = 2_000_000
OUT_COLS = 16
CHUNK = 400
GROUPS = CHUNK // 16
NCHUNKS = N_ROWS // CHUNK
NUM_CORES = 2
NUM_SUBCORES = 16
NW = NUM_CORES * NUM_SUBCORES
SLOTS = (NCHUNKS + NW - 1) // NW
FULL_SLOTS = NCHUNKS // NW
TAIL_W = NCHUNKS - FULL_SLOTS * NW
TAB_ROWS = 13


@functools.partial(
    pl.kernel,
    mesh=plsc.VectorSubcoreMesh(core_axis_name="c", subcore_axis_name="s"),
    out_type=jax.ShapeDtypeStruct((N_ROWS, OUT_COLS), jnp.float32),
    compiler_params=pltpu.CompilerParams(
        needs_layout_passes=False, use_tc_tiling_on_sc=True),
    scratch_types=[
        pltpu.VMEM((TAB_ROWS * 128,), jnp.float32),
        pltpu.VMEM((CHUNK * 4,), jnp.int32),
        pltpu.VMEM((CHUNK * 4,), jnp.int32),
        pltpu.VMEM((CHUNK, OUT_COLS), jnp.float32),
        pltpu.VMEM((CHUNK, OUT_COLS), jnp.float32),
        pltpu.SemaphoreType.DMA,
        pltpu.SemaphoreType.DMA,
        pltpu.SemaphoreType.DMA,
        pltpu.SemaphoreType.DMA,
    ],
)
def _sc_embed(note_hbm, tab_hbm, out_hbm, tab_v, in_a, in_b, out_a, out_b,
              in_sem_a, in_sem_b, out_sem_a, out_sem_b):
    wid = lax.axis_index("s") * NUM_CORES + lax.axis_index("c")
    pltpu.sync_copy(tab_hbm, tab_v)

    iota = lax.iota(jnp.int32, 16)
    iota4 = iota * 4
    ins = (in_a, in_b)
    outs = (out_a, out_b)
    in_sems = (in_sem_a, in_sem_b)
    out_sems = (out_sem_a, out_sem_b)
    csts = [jnp.full((16,), c, jnp.int32) for c in range(OUT_COLS)]

    def start_in(i, p):
        pltpu.async_copy(
            note_hbm.at[pl.ds((wid + i * NW) * CHUNK * 4, CHUNK * 4)],
            ins[p], in_sems[p])

    def wait_in(p):
        pltpu.make_async_copy(
            note_hbm.at[pl.ds(0, CHUNK * 4)], ins[p], in_sems[p]).wait()

    def start_out(i, p):
        pltpu.async_copy(
            outs[p], out_hbm.at[pl.ds((wid + i * NW) * CHUNK, CHUNK)],
            out_sems[p])

    def wait_out(p):
        pltpu.make_async_copy(
            outs[p], out_hbm.at[pl.ds(0, CHUNK)], out_sems[p]).wait()

    def compute(p):
        inv = ins[p]
        outv = outs[p]

        def group(j, carry):
            avec = j * 64 + iota4
            pos = plsc.load_gather(inv, [avec])
            pit = plsc.load_gather(inv, [avec + 1])
            lng = plsc.load_gather(inv, [avec + 2])
            vel = plsc.load_gather(inv, [avec + 3])
            cols = [plsc.load_gather(tab_v, [pit + (c * 128)])
                    for c in range(8)]
            cols.append(plsc.load_gather(tab_v, [vel + (10 * 128)]))
            cols.append(plsc.load_gather(tab_v, [lng + (11 * 128)]))
            cols.append(plsc.load_gather(tab_v, [pos + (8 * 128)]))
            cols.append(plsc.load_gather(tab_v, [pos + (9 * 128)]))
            pf = plsc.load_gather(tab_v, [pos + (12 * 128)])
            cols.extend([pf, pf, pf, pf])
            rvec = j * 16 + iota
            for c in range(OUT_COLS):
                plsc.store_scatter(outv, [rvec, csts[c]], cols[c])
            return carry

        lax.fori_loop(0, GROUPS, group, 0)

    start_in(0, 0)
    start_in(1, 1)
    wait_in(0)
    compute(0)
    start_out(0, 0)
    start_in(2, 0)
    wait_in(1)
    compute(1)
    start_out(1, 1)

    def pair(i2, carry):
        start_in(2 * i2 + 1, 1)
        wait_in(0)
        wait_out(0)
        compute(0)
        start_out(2 * i2, 0)

        @pl.when((2 * i2 + 2 < FULL_SLOTS) | (wid < TAIL_W))
        def _():
            start_in(2 * i2 + 2, 0)

        wait_in(1)
        wait_out(1)
        compute(1)
        start_out(2 * i2 + 1, 1)
        return carry

    lax.fori_loop(1, FULL_SLOTS // 2, pair, 0)

    @pl.when(wid < TAIL_W)
    def _():
        wait_in(0)
        wait_out(0)
        compute(0)
        start_out(FULL_SLOTS, 0)

    wait_out(0)
    wait_out(1)


def kernel(note_rep, length, resolution, time_sig_num, time_sig_den,
           note_table, octave_table):
    g = jnp.arange(128, dtype=jnp.int32)
    gf = g.astype(jnp.float32)
    nt = note_table[g % 12]
    oc = octave_table[jnp.clip(g // 12 - 1, 0, octave_table.shape[0] - 1)]
    two_pi = 2.0 * math.pi
    w_beat = two_pi / resolution
    w_measure = two_pi / (resolution * (time_sig_num / (time_sig_den / 4)))
    w_melody = two_pi / (resolution * length)
    l_sin = jnp.sin(jnp.sin(jnp.sin(gf * w_beat) * w_measure) * w_melody)
    l_cos = jnp.cos(jnp.cos(jnp.cos(gf * w_beat) * w_measure) * w_melody)
    l_vel = (g / 127).astype(jnp.float32)
    l_len = (g / length).astype(jnp.float32)
    tab = jnp.concatenate(
        [nt.T, oc.T, l_sin[None], l_cos[None], l_vel[None], l_len[None],
         gf[None]], axis=0)
    return _sc_embed(note_rep.reshape(-1), tab.reshape(-1))


# speedup vs baseline: 15.1821x; 13.5396x over previous
import functools
import math

import jax
import jax.numpy as jnp
from jax import lax
from jax.experimental import pallas as pl
from jax.experimental.pallas import tpu as pltpu
from jax.experimental.pallas import tpu_sc as plsc

N_ROWS = 2_000_000
OUT_COLS = 16
N_BLOCKS = N_ROWS // 128
CB = 5
CHUNK = CB * 128
NCHUNKS = N_BLOCKS // CB
NUM_CORES = 2
NUM_SUBCORES = 16
NW = NUM_CORES * NUM_SUBCORES
FULL_SLOTS = NCHUNKS // NW
TAIL_W = NCHUNKS - FULL_SLOTS * NW
TAB_ROWS = 13
OUT_HALF = CB * 1024


@functools.partial(
    pl.kernel,
    mesh=plsc.VectorSubcoreMesh(core_axis_name="c", subcore_axis_name="s"),
    out_type=jax.ShapeDtypeStruct((N_ROWS * OUT_COLS,), jnp.float32),
    compiler_params=pltpu.CompilerParams(
        needs_layout_passes=False, use_tc_tiling_on_sc=True),
    scratch_types=[
        pltpu.VMEM((TAB_ROWS * 128,), jnp.float32),
        pltpu.VMEM((CHUNK * 4,), jnp.int32),
        pltpu.VMEM((CHUNK * 4,), jnp.int32),
        pltpu.VMEM((CHUNK * OUT_COLS,), jnp.float32),
        pltpu.VMEM((CHUNK * OUT_COLS,), jnp.float32),
        pltpu.SemaphoreType.DMA,
        pltpu.SemaphoreType.DMA,
        pltpu.SemaphoreType.DMA,
        pltpu.SemaphoreType.DMA,
    ],
)
def _sc_embed(note_hbm, tab_hbm, out_hbm, tab_v, in_a, in_b, out_a, out_b,
              in_sem_a, in_sem_b, out_sem_a, out_sem_b):
    wid = lax.axis_index("s") * NUM_CORES + lax.axis_index("c")
    pltpu.sync_copy(tab_hbm, tab_v)

    ins = (in_a, in_b)
    outs = (out_a, out_b)
    in_sems = (in_sem_a, in_sem_b)
    out_sems = (out_sem_a, out_sem_b)

    def start_in(i, p):
        pltpu.async_copy(
            note_hbm.at[pl.ds((wid + i * NW) * CHUNK * 4, CHUNK * 4)],
            ins[p], in_sems[p])

    def wait_in(p):
        pltpu.make_async_copy(
            note_hbm.at[pl.ds(0, CHUNK * 4)], ins[p], in_sems[p]).wait()

    def start_out(i, p):
        b0 = (wid + i * NW) * CB
        for h in range(2):
            pltpu.async_copy(
                outs[p].at[pl.ds(h * OUT_HALF, OUT_HALF)],
                out_hbm.at[pl.ds((h * N_BLOCKS + b0) * 1024, OUT_HALF)],
                out_sems[p])

    def wait_out(p):
        for _ in range(2):
            pltpu.make_async_copy(
                outs[p].at[pl.ds(0, OUT_HALF)],
                out_hbm.at[pl.ds(0, OUT_HALF)], out_sems[p]).wait()

    def compute(p):
        inv = ins[p]
        outv = outs[p]

        def block(bb, carry):
            ibase = bb * 512
            obase = bb * 1024
            for j in range(8):
                l0 = j * 16
                pos = inv[pl.ds(ibase + l0, 16)]
                pit = inv[pl.ds(ibase + 128 + l0, 16)]
                lng = inv[pl.ds(ibase + 256 + l0, 16)]
                vel = inv[pl.ds(ibase + 384 + l0, 16)]
                cols = [plsc.load_gather(tab_v, [pit + (c * 128)])
                        for c in range(8)]
                cols.append(plsc.load_gather(tab_v, [vel + (10 * 128)]))
                cols.append(plsc.load_gather(tab_v, [lng + (11 * 128)]))
                cols.append(plsc.load_gather(tab_v, [pos + (8 * 128)]))
                cols.append(plsc.load_gather(tab_v, [pos + (9 * 128)]))
                pf = plsc.load_gather(tab_v, [pos + (12 * 128)])
                cols.extend([pf, pf, pf, pf])
                for c in range(OUT_COLS):
                    h, s = divmod(c, 8)
                    outv[pl.ds(h * OUT_HALF + obase + s * 128 + l0, 16)] = (
                        cols[c])
            return carry

        lax.fori_loop(0, CB, block, 0)

    start_in(0, 0)
    start_in(1, 1)
    wait_in(0)
    compute(0)
    start_out(0, 0)
    start_in(2, 0)
    wait_in(1)
    compute(1)
    start_out(1, 1)

    def pair(i2, carry):
        @pl.when((2 * i2 + 1 < FULL_SLOTS) | (wid < TAIL_W))
        def _():
            start_in(2 * i2 + 1, 1)

        wait_in(0)
        wait_out(0)
        compute(0)
        start_out(2 * i2, 0)

        @pl.when(2 * i2 + 2 < FULL_SLOTS)
        def _():
            start_in(2 * i2 + 2, 0)

        @pl.when((2 * i2 + 1 < FULL_SLOTS) | (wid < TAIL_W))
        def _():
            wait_in(1)
            wait_out(1)
            compute(1)
            start_out(2 * i2 + 1, 1)

        return carry

    lax.fori_loop(1, (FULL_SLOTS + 1) // 2, pair, 0)

    wait_out(0)
    wait_out(1)


def kernel(note_rep, length, resolution, time_sig_num, time_sig_den,
           note_table, octave_table):
    g = jnp.arange(128, dtype=jnp.int32)
    gf = g.astype(jnp.float32)
    nt = note_table[g % 12]
    oc = octave_table[jnp.clip(g // 12 - 1, 0, octave_table.shape[0] - 1)]
    two_pi = 2.0 * math.pi
    w_beat = two_pi / resolution
    w_measure = two_pi / (resolution * (time_sig_num / (time_sig_den / 4)))
    w_melody = two_pi / (resolution * length)
    l_sin = jnp.sin(jnp.sin(jnp.sin(gf * w_beat) * w_measure) * w_melody)
    l_cos = jnp.cos(jnp.cos(jnp.cos(gf * w_beat) * w_measure) * w_melody)
    l_vel = (g / 127).astype(jnp.float32)
    l_len = (g / length).astype(jnp.float32)
    tab = jnp.concatenate(
        [nt.T, oc.T, l_sin[None], l_cos[None], l_vel[None], l_len[None],
         gf[None]], axis=0)
    note_flat = note_rep.reshape(N_BLOCKS, 128, 4).transpose(0, 2, 1)
    res = _sc_embed(note_flat.reshape(-1), tab.reshape(-1))
    return res.reshape(2, N_BLOCKS, 8, 128).transpose(1, 3, 0, 2).reshape(
        N_ROWS, OUT_COLS)


# speedup vs baseline: 31.6378x; 2.0839x over previous
import functools
import math

import jax
import jax.numpy as jnp
from jax import lax
from jax.experimental import pallas as pl
from jax.experimental.pallas import tpu as pltpu
from jax.experimental.pallas import tpu_sc as plsc

N_ROWS = 2_000_000
OUT_COLS = 16
N_BLOCKS = N_ROWS // 128
CB = 5
CHUNK = CB * 128
NCHUNKS = N_BLOCKS // CB
NUM_CORES = 2
NUM_SUBCORES = 16
NW = NUM_CORES * NUM_SUBCORES
FULL_SLOTS = NCHUNKS // NW
TAIL_W = NCHUNKS - FULL_SLOTS * NW
TAB_ROWS = 13
OUT_HALF = CB * 1024


@functools.partial(
    pl.kernel,
    mesh=plsc.VectorSubcoreMesh(core_axis_name="c", subcore_axis_name="s"),
    out_type=jax.ShapeDtypeStruct((N_ROWS * OUT_COLS,), jnp.float32),
    compiler_params=pltpu.CompilerParams(
        needs_layout_passes=False, use_tc_tiling_on_sc=True),
    scratch_types=[
        pltpu.VMEM((TAB_ROWS * 128,), jnp.float32),
        pltpu.VMEM((CB, 4, 128), jnp.int32),
        pltpu.VMEM((CB, 4, 128), jnp.int32),
        pltpu.VMEM((CHUNK * OUT_COLS,), jnp.float32),
        pltpu.VMEM((CHUNK * OUT_COLS,), jnp.float32),
        pltpu.SemaphoreType.DMA,
        pltpu.SemaphoreType.DMA,
        pltpu.SemaphoreType.DMA,
        pltpu.SemaphoreType.DMA,
    ],
)
def _sc_embed(note_hbm, tab_hbm, out_hbm, tab_v, in_a, in_b, out_a, out_b,
              in_sem_a, in_sem_b, out_sem_a, out_sem_b):
    wid = lax.axis_index("s") * NUM_CORES + lax.axis_index("c")
    pltpu.sync_copy(tab_hbm, tab_v)

    ins = (in_a, in_b)
    outs = (out_a, out_b)
    in_sems = (in_sem_a, in_sem_b)
    out_sems = (out_sem_a, out_sem_b)

    def start_in(i, p):
        pltpu.async_copy(
            note_hbm.at[pl.ds((wid + i * NW) * CB, CB)],
            ins[p], in_sems[p])

    def wait_in(p):
        pltpu.make_async_copy(
            note_hbm.at[pl.ds(0, CB)], ins[p], in_sems[p]).wait()

    def start_out(i, p):
        b0 = (wid + i * NW) * CB
        for h in range(2):
            pltpu.async_copy(
                outs[p].at[pl.ds(h * OUT_HALF, OUT_HALF)],
                out_hbm.at[pl.ds((h * N_BLOCKS + b0) * 1024, OUT_HALF)],
                out_sems[p])

    def wait_out(p):
        for _ in range(2):
            pltpu.make_async_copy(
                outs[p].at[pl.ds(0, OUT_HALF)],
                out_hbm.at[pl.ds(0, OUT_HALF)], out_sems[p]).wait()

    def compute(p):
        inv = ins[p]
        outv = outs[p]

        def block(bb, carry):
            obase = bb * 1024
            for j in range(8):
                l0 = j * 16
                pos = inv[bb, 0, pl.ds(l0, 16)]
                pit = inv[bb, 1, pl.ds(l0, 16)]
                lng = inv[bb, 2, pl.ds(l0, 16)]
                vel = inv[bb, 3, pl.ds(l0, 16)]
                cols = [plsc.load_gather(tab_v, [pit + (c * 128)])
                        for c in range(8)]
                cols.append(plsc.load_gather(tab_v, [vel + (10 * 128)]))
                cols.append(plsc.load_gather(tab_v, [lng + (11 * 128)]))
                cols.append(plsc.load_gather(tab_v, [pos + (8 * 128)]))
                cols.append(plsc.load_gather(tab_v, [pos + (9 * 128)]))
                pf = plsc.load_gather(tab_v, [pos + (12 * 128)])
                cols.extend([pf, pf, pf, pf])
                for c in range(OUT_COLS):
                    h, s = divmod(c, 8)
                    outv[pl.ds(h * OUT_HALF + obase + s * 128 + l0, 16)] = (
                        cols[c])
            return carry

        lax.fori_loop(0, CB, block, 0)

    start_in(0, 0)
    start_in(1, 1)
    wait_in(0)
    compute(0)
    start_out(0, 0)
    start_in(2, 0)
    wait_in(1)
    compute(1)
    start_out(1, 1)

    def pair(i2, carry):
        @pl.when((2 * i2 + 1 < FULL_SLOTS) | (wid < TAIL_W))
        def _():
            start_in(2 * i2 + 1, 1)

        wait_in(0)
        wait_out(0)
        compute(0)
        start_out(2 * i2, 0)

        @pl.when(2 * i2 + 2 < FULL_SLOTS)
        def _():
            start_in(2 * i2 + 2, 0)

        @pl.when((2 * i2 + 1 < FULL_SLOTS) | (wid < TAIL_W))
        def _():
            wait_in(1)
            wait_out(1)
            compute(1)
            start_out(2 * i2 + 1, 1)

        return carry

    lax.fori_loop(1, (FULL_SLOTS + 1) // 2, pair, 0)

    wait_out(0)
    wait_out(1)


def kernel(note_rep, length, resolution, time_sig_num, time_sig_den,
           note_table, octave_table):
    g = jnp.arange(128, dtype=jnp.int32)
    gf = g.astype(jnp.float32)
    nt = note_table[g % 12]
    oc = octave_table[jnp.clip(g // 12 - 1, 0, octave_table.shape[0] - 1)]
    two_pi = 2.0 * math.pi
    w_beat = two_pi / resolution
    w_measure = two_pi / (resolution * (time_sig_num / (time_sig_den / 4)))
    w_melody = two_pi / (resolution * length)
    l_sin = jnp.sin(jnp.sin(jnp.sin(gf * w_beat) * w_measure) * w_melody)
    l_cos = jnp.cos(jnp.cos(jnp.cos(gf * w_beat) * w_measure) * w_melody)
    l_vel = (g / 127).astype(jnp.float32)
    l_len = (g / length).astype(jnp.float32)
    tab = jnp.concatenate(
        [nt.T, oc.T, l_sin[None], l_cos[None], l_vel[None], l_len[None],
         gf[None]], axis=0)
    note_view = note_rep.reshape(N_BLOCKS, 128, 4).transpose(0, 2, 1)
    res = _sc_embed(note_view, tab.reshape(-1))
    return res.reshape(2, N_BLOCKS, 8, 128).transpose(1, 3, 0, 2).reshape(
        N_ROWS, OUT_COLS)
